# baseline (speedup 1.0000x reference)
import functools

import jax
import jax.numpy as jnp
from jax import lax
from jax.experimental import pallas as pl
from jax.experimental.pallas import tpu as pltpu
from jax.experimental.pallas import tpu_sc as plsc

_BP = 64
_NW = 32


def _fused_body(top_ref, emb_ref, h_ref, c_ref,
                wih_ref, whh_ref, b_ref,
                oh_ref, oc_ref, ox_ref,
                hq_ref, cq_ref, nhq_ref, ncq_ref):
    BP = _BP
    H = emb_ref.shape[1]
    base = pl.program_id(0) * BP

    for j in range(BP):
        t = top_ref[base + j]
        src = j * 132 + t * 4
        hq_ref[pl.ds(4 * j, 4), :] = h_ref[pl.ds(src, 4), :]
        cq_ref[pl.ds(4 * j, 4), :] = c_ref[pl.ds(src, 4), :]

    h_prev = []
    c_prev = []
    for l in range(2):
        h_prev.append(jnp.concatenate(
            [hq_ref[pl.Slice(l, BP, 4), :],
             hq_ref[pl.Slice(2 + l, BP, 4), :]], axis=1))
        c_prev.append(jnp.concatenate(
            [cq_ref[pl.Slice(l, BP, 4), :],
             cq_ref[pl.Slice(2 + l, BP, 4), :]], axis=1))

    wih = wih_ref[...]
    whh = whh_ref[...]
    bias = b_ref[...]
    x = emb_ref[...]
    h_new = []
    c_new = []
    for l in range(2):
        gates = (jax.lax.dot_general(x, wih[l], (((1,), (1,)), ((), ())))
                 + jax.lax.dot_general(h_prev[l], whh[l],
                                       (((1,), (1,)), ((), ())))
                 + bias[l:l + 1, :])
        i_g = jax.nn.sigmoid(gates[:, 0 * H:1 * H])
        f_g = jax.nn.sigmoid(gates[:, 1 * H:2 * H])
        g_g = jnp.tanh(gates[:, 2 * H:3 * H])
        o_g = jax.nn.sigmoid(gates[:, 3 * H:4 * H])
        c_l = f_g * c_prev[l] + i_g * g_g
        h_l = o_g * jnp.tanh(c_l)
        h_new.append(h_l)
        c_new.append(c_l)
        x = h_l

    for q, (hs, cs) in enumerate(
            [(h_new[0][:, :128], c_new[0][:, :128]),
             (h_new[1][:, :128], c_new[1][:, :128]),
             (h_new[0][:, 128:], c_new[0][:, 128:]),
             (h_new[1][:, 128:], c_new[1][:, 128:])]):
        nhq_ref[pl.Slice(q, BP, 4), :] = hs
        ncq_ref[pl.Slice(q, BP, 4), :] = cs

    oh_ref[...] = h_ref[...]
    oc_ref[...] = c_ref[...]
    for j in range(BP):
        t = top_ref[base + j]
        dst = j * 132 + (t + 1) * 4
        oh_ref[pl.ds(dst, 4), :] = nhq_ref[pl.ds(4 * j, 4), :]
        oc_ref[pl.ds(dst, 4), :] = ncq_ref[pl.ds(4 * j, 4), :]
    ox_ref[...] = x


def _trees_sc_body(tv_ref, top_ref, embv_ref, ot_ref,
                   top_v, emb_v, idx_v, sem):
    i32 = jnp.int32
    w = lax.axis_index("s") * 2 + lax.axis_index("c")
    rows_per_w = 4096
    beams_per_w = 64

    pltpu.sync_copy(tv_ref.at[pl.ds(w * rows_per_w, rows_per_w)],
                    ot_ref.at[pl.ds(w * rows_per_w, rows_per_w)])

    pltpu.sync_copy(embv_ref.at[pl.ds(w * 128, 128)], emb_v)
    for b in range(8):
        src = top_ref.at[pl.ds(w * beams_per_w + b * 8, 8)]
        pltpu.sync_copy(src, top_v.at[pl.ds(b * 16, 8)])
        pltpu.sync_copy(src, top_v.at[pl.ds(b * 16 + 8, 8)])

    iota = lax.iota(i32, 16)
    jmod = iota & 7
    half8 = (iota >> 3) * 8
    for b in range(8):
        t16 = top_v[pl.ds(b * 16, 16)]
        j16 = b * 8 + jmod
        dst16 = ((w * beams_per_w + j16) * 64
                 + (t16 >> 3) * 16 + (t16 & 7) + half8)
        idx_v[pl.ds(b * 16, 16)] = dst16

    pltpu.async_copy(emb_v, ot_ref.at[idx_v], sem).wait()


def kernel(hiddens, cells, trees, top_position, shifted_embs,
           W_ih, W_hh, b_ih, b_hh):
    P, S1, H, L = hiddens.shape
    S, D = trees.shape[1], trees.shape[2]
    G = 4 * H
    f32 = jnp.float32
    BP = _BP

    hv = hiddens.reshape(P, S1, 2, 128, 2).transpose(0, 1, 2, 4, 3) \
                .reshape(P * S1 * 4, 128)
    cv = cells.reshape(P, S1, 2, 128, 2).transpose(0, 1, 2, 4, 3) \
              .reshape(P * S1 * 4, 128)
    tv = trees.reshape(P, 4, 8, 2, 128).transpose(0, 1, 3, 2, 4) \
              .reshape(P * 64, 128)
    embv = shifted_embs.reshape(P // 8, 8, 2, 128).transpose(0, 2, 1, 3) \
                       .reshape(P * 2, 128)
    top = top_position.astype(jnp.int32)
    bias = (b_ih + b_hh).astype(f32)

    trees_sc = functools.partial(
        pl.kernel,
        out_type=jax.ShapeDtypeStruct((P * 64, 128), f32),
        mesh=plsc.VectorSubcoreMesh(core_axis_name="c", subcore_axis_name="s"),
        scratch_types=[
            pltpu.VMEM((128,), jnp.int32),
            pltpu.VMEM((128, 128), f32),
            pltpu.VMEM((128,), jnp.int32),
            pltpu.SemaphoreType.DMA,
        ],
    )(_trees_sc_body)
    ot = trees_sc(tv, top, embv)

    grid = (P // BP,)
    out_shapes = [
        jax.ShapeDtypeStruct((P * S1 * 4, 128), f32),
        jax.ShapeDtypeStruct((P * S1 * 4, 128), f32),
        jax.ShapeDtypeStruct((P, H), f32),
    ]
    in_specs = [
        pl.BlockSpec(memory_space=pltpu.SMEM),
        pl.BlockSpec((BP, D), lambda i: (i, 0)),
        pl.BlockSpec((BP * 132, 128), lambda i: (i, 0)),
        pl.BlockSpec((BP * 132, 128), lambda i: (i, 0)),
        pl.BlockSpec((L, G, D), lambda i: (0, 0, 0)),
        pl.BlockSpec((L, G, H), lambda i: (0, 0, 0)),
        pl.BlockSpec((L, G), lambda i: (0, 0)),
    ]
    out_specs = [
        pl.BlockSpec((BP * 132, 128), lambda i: (i, 0)),
        pl.BlockSpec((BP * 132, 128), lambda i: (i, 0)),
        pl.BlockSpec((BP, H), lambda i: (i, 0)),
    ]

    oh, oc, ox = pl.pallas_call(
        _fused_body,
        grid=grid,
        in_specs=in_specs,
        out_specs=out_specs,
        out_shape=out_shapes,
        scratch_shapes=[pltpu.VMEM((4 * BP, 128), f32),
                        pltpu.VMEM((4 * BP, 128), f32),
                        pltpu.VMEM((4 * BP, 128), f32),
                        pltpu.VMEM((4 * BP, 128), f32)],
    )(top, shifted_embs, hv, cv, W_ih, W_hh, bias)

    new_hiddens = oh.reshape(P, S1, 2, 2, 128).transpose(0, 1, 2, 4, 3) \
                    .reshape(P, S1, H, L)
    new_cells = oc.reshape(P, S1, 2, 2, 128).transpose(0, 1, 2, 4, 3) \
                  .reshape(P, S1, H, L)
    new_trees = ot.reshape(P, 4, 2, 8, 128).transpose(0, 1, 3, 2, 4) \
                  .reshape(P, S, D)
    return (new_hiddens, new_cells, new_trees, ox)


# speedup vs baseline: 8.6625x; 8.6625x over previous
import functools

import jax
import jax.numpy as jnp
from jax import lax
from jax.experimental import pallas as pl
from jax.experimental.pallas import tpu as pltpu
from jax.experimental.pallas import tpu_sc as plsc

_BP = 64
_NW = 32


def _fused_body(top_ref, emb_ref, h_ref, c_ref,
                wih_ref, whh_ref, b_ref,
                oh_ref, oc_ref, ox_ref,
                hq_ref, cq_ref, nhq_ref, ncq_ref):
    BP = _BP
    H = emb_ref.shape[1]
    base = pl.program_id(0) * BP

    for j in range(BP):
        t = top_ref[base + j]
        src = j * 132 + t * 4
        hq_ref[pl.ds(4 * j, 4), :] = h_ref[pl.ds(src, 4), :]
        cq_ref[pl.ds(4 * j, 4), :] = c_ref[pl.ds(src, 4), :]

    h_prev = []
    c_prev = []
    for l in range(2):
        h_prev.append(jnp.concatenate(
            [hq_ref[pl.Slice(l, BP, 4), :],
             hq_ref[pl.Slice(2 + l, BP, 4), :]], axis=1))
        c_prev.append(jnp.concatenate(
            [cq_ref[pl.Slice(l, BP, 4), :],
             cq_ref[pl.Slice(2 + l, BP, 4), :]], axis=1))

    wih = wih_ref[...]
    whh = whh_ref[...]
    bias = b_ref[...]
    x = emb_ref[...]
    h_new = []
    c_new = []
    for l in range(2):
        gates = (jax.lax.dot_general(x, wih[l], (((1,), (1,)), ((), ())))
                 + jax.lax.dot_general(h_prev[l], whh[l],
                                       (((1,), (1,)), ((), ())))
                 + bias[l:l + 1, :])
        i_g = jax.nn.sigmoid(gates[:, 0 * H:1 * H])
        f_g = jax.nn.sigmoid(gates[:, 1 * H:2 * H])
        g_g = jnp.tanh(gates[:, 2 * H:3 * H])
        o_g = jax.nn.sigmoid(gates[:, 3 * H:4 * H])
        c_l = f_g * c_prev[l] + i_g * g_g
        h_l = o_g * jnp.tanh(c_l)
        h_new.append(h_l)
        c_new.append(c_l)
        x = h_l

    for q, (hs, cs) in enumerate(
            [(h_new[0][:, :128], c_new[0][:, :128]),
             (h_new[1][:, :128], c_new[1][:, :128]),
             (h_new[0][:, 128:], c_new[0][:, 128:]),
             (h_new[1][:, 128:], c_new[1][:, 128:])]):
        nhq_ref[pl.Slice(q, BP, 4), :] = hs
        ncq_ref[pl.Slice(q, BP, 4), :] = cs

    oh_ref[...] = h_ref[...]
    oc_ref[...] = c_ref[...]
    for j in range(BP):
        t = top_ref[base + j]
        dst = j * 132 + (t + 1) * 4
        oh_ref[pl.ds(dst, 4), :] = nhq_ref[pl.ds(4 * j, 4), :]
        oc_ref[pl.ds(dst, 4), :] = ncq_ref[pl.ds(4 * j, 4), :]
    ox_ref[...] = x


def _trees_sc_body(tv_ref, top_ref, embv_ref, ot_ref,
                   top_v, emb_v, idx_v, bufs, csem, sem):
    i32 = jnp.int32
    w = lax.axis_index("s") * 2 + lax.axis_index("c")
    rows_per_w = 4096
    beams_per_w = 64

    NB = 4
    CH = 128
    NCH = rows_per_w // CH
    base_row = w * rows_per_w
    rd = [None] * NB
    wr = [None] * NB
    for b in range(NB):
        rd[b] = pltpu.make_async_copy(
            tv_ref.at[pl.ds(base_row + b * CH, CH)], bufs.at[b], csem.at[b])
        rd[b].start()
    for i in range(NCH):
        b = i % NB
        rd[b].wait()
        wr[b] = pltpu.make_async_copy(
            bufs.at[b], ot_ref.at[pl.ds(base_row + i * CH, CH)], csem.at[b])
        wr[b].start()
        nxt = i + NB
        if nxt < NCH:
            wr[b].wait()
            rd[b] = pltpu.make_async_copy(
                tv_ref.at[pl.ds(base_row + nxt * CH, CH)], bufs.at[b],
                csem.at[b])
            rd[b].start()
    for i in range(NCH - NB, NCH):
        wr[i % NB].wait()

    pltpu.sync_copy(embv_ref.at[pl.ds(w * 128, 128)], emb_v)
    for b in range(8):
        src = top_ref.at[pl.ds(w * beams_per_w + b * 8, 8)]
        pltpu.sync_copy(src, top_v.at[pl.ds(b * 16, 8)])
        pltpu.sync_copy(src, top_v.at[pl.ds(b * 16 + 8, 8)])

    iota = lax.iota(i32, 16)
    jmod = iota & 7
    half8 = (iota >> 3) * 8
    for b in range(8):
        t16 = top_v[pl.ds(b * 16, 16)]
        j16 = b * 8 + jmod
        dst16 = ((w * beams_per_w + j16) * 64
                 + (t16 >> 3) * 16 + (t16 & 7) + half8)
        idx_v[pl.ds(b * 16, 16)] = dst16

    pltpu.async_copy(emb_v, ot_ref.at[idx_v], sem).wait()


def kernel(hiddens, cells, trees, top_position, shifted_embs,
           W_ih, W_hh, b_ih, b_hh):
    P, S1, H, L = hiddens.shape
    S, D = trees.shape[1], trees.shape[2]
    G = 4 * H
    f32 = jnp.float32
    BP = _BP

    hv = hiddens.reshape(P, S1, 2, 128, 2).transpose(0, 1, 2, 4, 3) \
                .reshape(P * S1 * 4, 128)
    cv = cells.reshape(P, S1, 2, 128, 2).transpose(0, 1, 2, 4, 3) \
              .reshape(P * S1 * 4, 128)
    tv = trees.reshape(P, 4, 8, 2, 128).transpose(0, 1, 3, 2, 4) \
              .reshape(P * 64, 128)
    embv = shifted_embs.reshape(P // 8, 8, 2, 128).transpose(0, 2, 1, 3) \
                       .reshape(P * 2, 128)
    top = top_position.astype(jnp.int32)
    bias = (b_ih + b_hh).astype(f32)

    trees_sc = functools.partial(
        pl.kernel,
        out_type=jax.ShapeDtypeStruct((P * 64, 128), f32),
        mesh=plsc.VectorSubcoreMesh(core_axis_name="c", subcore_axis_name="s"),
        scratch_types=[
            pltpu.VMEM((128,), jnp.int32),
            pltpu.VMEM((128, 128), f32),
            pltpu.VMEM((128,), jnp.int32),
            pltpu.VMEM((4, 128, 128), f32),
            pltpu.SemaphoreType.DMA((4,)),
            pltpu.SemaphoreType.DMA,
        ],
    )(_trees_sc_body)
    ot = trees_sc(tv, top, embv)

    grid = (P // BP,)
    out_shapes = [
        jax.ShapeDtypeStruct((P * S1 * 4, 128), f32),
        jax.ShapeDtypeStruct((P * S1 * 4, 128), f32),
        jax.ShapeDtypeStruct((P, H), f32),
    ]
    in_specs = [
        pl.BlockSpec(memory_space=pltpu.SMEM),
        pl.BlockSpec((BP, D), lambda i: (i, 0)),
        pl.BlockSpec((BP * 132, 128), lambda i: (i, 0)),
        pl.BlockSpec((BP * 132, 128), lambda i: (i, 0)),
        pl.BlockSpec((L, G, D), lambda i: (0, 0, 0)),
        pl.BlockSpec((L, G, H), lambda i: (0, 0, 0)),
        pl.BlockSpec((L, G), lambda i: (0, 0)),
    ]
    out_specs = [
        pl.BlockSpec((BP * 132, 128), lambda i: (i, 0)),
        pl.BlockSpec((BP * 132, 128), lambda i: (i, 0)),
        pl.BlockSpec((BP, H), lambda i: (i, 0)),
    ]

    oh, oc, ox = pl.pallas_call(
        _fused_body,
        grid=grid,
        in_specs=in_specs,
        out_specs=out_specs,
        out_shape=out_shapes,
        scratch_shapes=[pltpu.VMEM((4 * BP, 128), f32),
                        pltpu.VMEM((4 * BP, 128), f32),
                        pltpu.VMEM((4 * BP, 128), f32),
                        pltpu.VMEM((4 * BP, 128), f32)],
    )(top, shifted_embs, hv, cv, W_ih, W_hh, bias)

    new_hiddens = oh.reshape(P, S1, 2, 2, 128).transpose(0, 1, 2, 4, 3) \
                    .reshape(P, S1, H, L)
    new_cells = oc.reshape(P, S1, 2, 2, 128).transpose(0, 1, 2, 4, 3) \
                  .reshape(P, S1, H, L)
    new_trees = ot.reshape(P, 4, 2, 8, 128).transpose(0, 1, 3, 2, 4) \
                  .reshape(P, S, D)
    return (new_hiddens, new_cells, new_trees, ox)


# speedup vs baseline: 9.3542x; 1.0799x over previous
import jax
import jax.numpy as jnp
from jax.experimental import pallas as pl
from jax.experimental.pallas import tpu as pltpu

_BP = 32


def _fused_body(top_ref, emb_ref, h_ref, c_ref, t_ref,
                wih_ref, whh_ref, b_ref,
                oh_ref, oc_ref, ot_ref, ox_ref,
                hq_ref, cq_ref, nhq_ref, ncq_ref):
    f32 = jnp.float32
    BP = _BP
    H = emb_ref.shape[1]
    base = pl.program_id(0) * BP

    for j in range(BP):
        t = top_ref[base + j]
        src = j * 132 + t * 4
        hq_ref[pl.ds(4 * j, 4), :] = h_ref[pl.ds(src, 4), :]
        cq_ref[pl.ds(4 * j, 4), :] = c_ref[pl.ds(src, 4), :]

    h_prev = []
    c_prev = []
    for l in range(2):
        h_prev.append(jnp.concatenate(
            [hq_ref[pl.Slice(l, BP, 4), :],
             hq_ref[pl.Slice(2 + l, BP, 4), :]], axis=1))
        c_prev.append(jnp.concatenate(
            [cq_ref[pl.Slice(l, BP, 4), :],
             cq_ref[pl.Slice(2 + l, BP, 4), :]], axis=1))

    wih = wih_ref[...]
    whh = whh_ref[...]
    bias = b_ref[...]
    x = emb_ref[...]
    h_new = []
    c_new = []
    for l in range(2):
        gates = (jax.lax.dot_general(x, wih[l], (((1,), (1,)), ((), ())))
                 + jax.lax.dot_general(h_prev[l], whh[l],
                                       (((1,), (1,)), ((), ())))
                 + bias[l:l + 1, :])
        i_g = jax.nn.sigmoid(gates[:, 0 * H:1 * H])
        f_g = jax.nn.sigmoid(gates[:, 1 * H:2 * H])
        g_g = jnp.tanh(gates[:, 2 * H:3 * H])
        o_g = jax.nn.sigmoid(gates[:, 3 * H:4 * H])
        c_l = f_g * c_prev[l] + i_g * g_g
        h_l = o_g * jnp.tanh(c_l)
        h_new.append(h_l)
        c_new.append(c_l)
        x = h_l

    for q, (hs, cs) in enumerate(
            [(h_new[0][:, :128], c_new[0][:, :128]),
             (h_new[1][:, :128], c_new[1][:, :128]),
             (h_new[0][:, 128:], c_new[0][:, 128:]),
             (h_new[1][:, 128:], c_new[1][:, 128:])]):
        nhq_ref[pl.Slice(q, BP, 4), :] = hs
        ncq_ref[pl.Slice(q, BP, 4), :] = cs

    oh_ref[...] = h_ref[...]
    oc_ref[...] = c_ref[...]
    ot_ref[...] = t_ref[...]
    emb = emb_ref[...]
    for j in range(BP):
        t = top_ref[base + j]
        dst = j * 132 + (t + 1) * 4
        oh_ref[pl.ds(dst, 4), :] = nhq_ref[pl.ds(4 * j, 4), :]
        oc_ref[pl.ds(dst, 4), :] = ncq_ref[pl.ds(4 * j, 4), :]
        r0 = j * 64 + (t >> 3) * 16 + (t & 7)
        ot_ref[pl.ds(r0, 1), :] = emb[j:j + 1, :128]
        ot_ref[pl.ds(r0 + 8, 1), :] = emb[j:j + 1, 128:]
    ox_ref[...] = x


def kernel(hiddens, cells, trees, top_position, shifted_embs,
           W_ih, W_hh, b_ih, b_hh):
    P, S1, H, L = hiddens.shape
    S, D = trees.shape[1], trees.shape[2]
    G = 4 * H
    f32 = jnp.float32
    BP = _BP

    hv = hiddens.reshape(P, S1, 2, 128, 2).transpose(0, 1, 2, 4, 3) \
                .reshape(P * S1 * 4, 128)
    cv = cells.reshape(P, S1, 2, 128, 2).transpose(0, 1, 2, 4, 3) \
              .reshape(P * S1 * 4, 128)
    tv = trees.reshape(P, 4, 8, 2, 128).transpose(0, 1, 3, 2, 4) \
              .reshape(P * 64, 128)
    top = top_position.astype(jnp.int32)
    bias = (b_ih + b_hh).astype(f32)

    grid = (P // BP,)
    out_shapes = [
        jax.ShapeDtypeStruct((P * S1 * 4, 128), f32),
        jax.ShapeDtypeStruct((P * S1 * 4, 128), f32),
        jax.ShapeDtypeStruct((P * 64, 128), f32),
        jax.ShapeDtypeStruct((P, H), f32),
    ]
    in_specs = [
        pl.BlockSpec(memory_space=pltpu.SMEM),
        pl.BlockSpec((BP, D), lambda i: (i, 0)),
        pl.BlockSpec((BP * 132, 128), lambda i: (i, 0)),
        pl.BlockSpec((BP * 132, 128), lambda i: (i, 0)),
        pl.BlockSpec((BP * 64, 128), lambda i: (i, 0)),
        pl.BlockSpec((L, G, D), lambda i: (0, 0, 0)),
        pl.BlockSpec((L, G, H), lambda i: (0, 0, 0)),
        pl.BlockSpec((L, G), lambda i: (0, 0)),
    ]
    out_specs = [
        pl.BlockSpec((BP * 132, 128), lambda i: (i, 0)),
        pl.BlockSpec((BP * 132, 128), lambda i: (i, 0)),
        pl.BlockSpec((BP * 64, 128), lambda i: (i, 0)),
        pl.BlockSpec((BP, H), lambda i: (i, 0)),
    ]

    oh, oc, ot, ox = pl.pallas_call(
        _fused_body,
        grid=grid,
        in_specs=in_specs,
        out_specs=out_specs,
        out_shape=out_shapes,
        scratch_shapes=[pltpu.VMEM((4 * BP, 128), f32),
                        pltpu.VMEM((4 * BP, 128), f32),
                        pltpu.VMEM((4 * BP, 128), f32),
                        pltpu.VMEM((4 * BP, 128), f32)],
    )(top, shifted_embs, hv, cv, tv, W_ih, W_hh, bias)

    new_hiddens = oh.reshape(P, S1, 2, 2, 128).transpose(0, 1, 2, 4, 3) \
                    .reshape(P, S1, H, L)
    new_cells = oc.reshape(P, S1, 2, 2, 128).transpose(0, 1, 2, 4, 3) \
                  .reshape(P, S1, H, L)
    new_trees = ot.reshape(P, 4, 2, 8, 128).transpose(0, 1, 3, 2, 4) \
                  .reshape(P, S, D)
    return (new_hiddens, new_cells, new_trees, ox)


# speedup vs baseline: 9.4621x; 1.0115x over previous
import jax
import jax.numpy as jnp
from jax.experimental import pallas as pl
from jax.experimental.pallas import tpu as pltpu

_BP = 64


def _fused_body(top_ref, emb_ref, h_ref, c_ref, t_ref,
                wih_ref, whh_ref, b_ref,
                oh_ref, oc_ref, ot_ref, ox_ref,
                hq_ref, cq_ref, nhq_ref, ncq_ref):
    f32 = jnp.float32
    BP = _BP
    H = emb_ref.shape[1]
    base = pl.program_id(0) * BP

    for j in range(BP):
        t = top_ref[base + j]
        src = j * 132 + t * 4
        hq_ref[pl.ds(4 * j, 4), :] = h_ref[pl.ds(src, 4), :]
        cq_ref[pl.ds(4 * j, 4), :] = c_ref[pl.ds(src, 4), :]

    h_prev = []
    c_prev = []
    for l in range(2):
        h_prev.append(jnp.concatenate(
            [hq_ref[pl.Slice(l, BP, 4), :],
             hq_ref[pl.Slice(2 + l, BP, 4), :]], axis=1))
        c_prev.append(jnp.concatenate(
            [cq_ref[pl.Slice(l, BP, 4), :],
             cq_ref[pl.Slice(2 + l, BP, 4), :]], axis=1))

    wih = wih_ref[...]
    whh = whh_ref[...]
    bias = b_ref[...]
    x = emb_ref[...]
    h_new = []
    c_new = []
    for l in range(2):
        gates = (jax.lax.dot_general(x, wih[l], (((1,), (1,)), ((), ())))
                 + jax.lax.dot_general(h_prev[l], whh[l],
                                       (((1,), (1,)), ((), ())))
                 + bias[l:l + 1, :])
        i_g = jax.nn.sigmoid(gates[:, 0 * H:1 * H])
        f_g = jax.nn.sigmoid(gates[:, 1 * H:2 * H])
        g_g = jnp.tanh(gates[:, 2 * H:3 * H])
        o_g = jax.nn.sigmoid(gates[:, 3 * H:4 * H])
        c_l = f_g * c_prev[l] + i_g * g_g
        h_l = o_g * jnp.tanh(c_l)
        h_new.append(h_l)
        c_new.append(c_l)
        x = h_l

    for q, (hs, cs) in enumerate(
            [(h_new[0][:, :128], c_new[0][:, :128]),
             (h_new[1][:, :128], c_new[1][:, :128]),
             (h_new[0][:, 128:], c_new[0][:, 128:]),
             (h_new[1][:, 128:], c_new[1][:, 128:])]):
        nhq_ref[pl.Slice(q, BP, 4), :] = hs
        ncq_ref[pl.Slice(q, BP, 4), :] = cs

    oh_ref[...] = h_ref[...]
    oc_ref[...] = c_ref[...]
    ot_ref[...] = t_ref[...]
    emb = emb_ref[...]
    for j in range(BP):
        t = top_ref[base + j]
        dst = j * 132 + (t + 1) * 4
        oh_ref[pl.ds(dst, 4), :] = nhq_ref[pl.ds(4 * j, 4), :]
        oc_ref[pl.ds(dst, 4), :] = ncq_ref[pl.ds(4 * j, 4), :]
        r0 = j * 64 + (t >> 3) * 16 + (t & 7)
        ot_ref[pl.ds(r0, 1), :] = emb[j:j + 1, :128]
        ot_ref[pl.ds(r0 + 8, 1), :] = emb[j:j + 1, 128:]
    ox_ref[...] = x


def kernel(hiddens, cells, trees, top_position, shifted_embs,
           W_ih, W_hh, b_ih, b_hh):
    P, S1, H, L = hiddens.shape
    S, D = trees.shape[1], trees.shape[2]
    G = 4 * H
    f32 = jnp.float32
    BP = _BP

    hv = hiddens.reshape(P, S1, 2, 128, 2).transpose(0, 1, 2, 4, 3) \
                .reshape(P * S1 * 4, 128)
    cv = cells.reshape(P, S1, 2, 128, 2).transpose(0, 1, 2, 4, 3) \
              .reshape(P * S1 * 4, 128)
    tv = trees.reshape(P, 4, 8, 2, 128).transpose(0, 1, 3, 2, 4) \
              .reshape(P * 64, 128)
    top = top_position.astype(jnp.int32)
    bias = (b_ih + b_hh).astype(f32)

    grid = (P // BP,)
    out_shapes = [
        jax.ShapeDtypeStruct((P * S1 * 4, 128), f32),
        jax.ShapeDtypeStruct((P * S1 * 4, 128), f32),
        jax.ShapeDtypeStruct((P * 64, 128), f32),
        jax.ShapeDtypeStruct((P, H), f32),
    ]
    in_specs = [
        pl.BlockSpec(memory_space=pltpu.SMEM),
        pl.BlockSpec((BP, D), lambda i: (i, 0)),
        pl.BlockSpec((BP * 132, 128), lambda i: (i, 0)),
        pl.BlockSpec((BP * 132, 128), lambda i: (i, 0)),
        pl.BlockSpec((BP * 64, 128), lambda i: (i, 0)),
        pl.BlockSpec((L, G, D), lambda i: (0, 0, 0)),
        pl.BlockSpec((L, G, H), lambda i: (0, 0, 0)),
        pl.BlockSpec((L, G), lambda i: (0, 0)),
    ]
    out_specs = [
        pl.BlockSpec((BP * 132, 128), lambda i: (i, 0)),
        pl.BlockSpec((BP * 132, 128), lambda i: (i, 0)),
        pl.BlockSpec((BP * 64, 128), lambda i: (i, 0)),
        pl.BlockSpec((BP, H), lambda i: (i, 0)),
    ]

    oh, oc, ot, ox = pl.pallas_call(
        _fused_body,
        grid=grid,
        in_specs=in_specs,
        out_specs=out_specs,
        out_shape=out_shapes,
        scratch_shapes=[pltpu.VMEM((4 * BP, 128), f32),
                        pltpu.VMEM((4 * BP, 128), f32),
                        pltpu.VMEM((4 * BP, 128), f32),
                        pltpu.VMEM((4 * BP, 128), f32)],
    )(top, shifted_embs, hv, cv, tv, W_ih, W_hh, bias)

    new_hiddens = oh.reshape(P, S1, 2, 2, 128).transpose(0, 1, 2, 4, 3) \
                    .reshape(P, S1, H, L)
    new_cells = oc.reshape(P, S1, 2, 2, 128).transpose(0, 1, 2, 4, 3) \
                  .reshape(P, S1, H, L)
    new_trees = ot.reshape(P, 4, 2, 8, 128).transpose(0, 1, 3, 2, 4) \
                  .reshape(P, S, D)
    return (new_hiddens, new_cells, new_trees, ox)
